# Initial kernel scaffold; baseline (speedup 1.0000x reference)
import jax
import jax.numpy as jnp
from jax.experimental import pallas as pl


def kernel(h, edge_index, snorm_n, W_fc, gamma, beta):
    raise NotImplementedError("write your pallas kernel here")


# speedup vs baseline: 3.5862x; 3.5862x over previous
import jax
import jax.numpy as jnp
from jax import lax
from jax.experimental import pallas as pl
from jax.experimental.pallas import tpu as pltpu
from jax.experimental.pallas import tpu_sc as plsc

N = 10000
E = 320000
D = 128
H = 64
EPS = 1e-5

NSUB = 16
EPT = E // NSUB
CHUNK = 80
NCH = EPT // CHUNK
TILES_IO = 10
RPT = N // TILES_IO
ZR = 40


def _matmul_body(h_ref, w_ref, zt_ref):
    z = lax.dot_general(
        h_ref[...], w_ref[...], (((1,), (1,)), ((), ())),
        preferred_element_type=jnp.float32, precision=lax.Precision.HIGHEST)
    zt_ref[0:N, :] = z[:, 0:H]
    zt_ref[N:2 * N, :] = z[:, H:D]


def _edge_body(zt_hbm, src_hbm, dst_hbm, out_hbm,
               acc_sh, zbuf, sidx, didx, sgi, dgi, zs, zd, contrib,
               sem0, sem1):
    c = lax.axis_index("c")
    s = lax.axis_index("s")

    @pl.when(s < TILES_IO)
    def _zero_acc():
        def zfill(r, carry):
            for k in range(D // 16):
                zbuf[r, pl.ds(16 * k, 16)] = jnp.zeros((16,), jnp.float32)
            return carry

        lax.fori_loop(0, ZR, zfill, 0)

        def zcopy(j, carry):
            pltpu.sync_copy(zbuf, acc_sh.at[pl.ds(s * RPT + j * ZR, ZR)])
            return carry

        lax.fori_loop(0, RPT // ZR, zcopy, 0)

    plsc.subcore_barrier()

    coff = c * N

    def chunk(i, carry):
        base = s * EPT + i * CHUNK
        pltpu.sync_copy(src_hbm.at[pl.ds(base, CHUNK)], sidx)
        pltpu.sync_copy(dst_hbm.at[pl.ds(base, CHUNK)], didx)
        for k in range(CHUNK // 16):
            sl = pl.ds(16 * k, 16)
            sgi[sl] = sidx[sl] + coff
            dgi[sl] = didx[sl] + coff
        cp_s = pltpu.async_copy(zt_hbm.at[sgi], zs, sem0)
        cp_d = pltpu.async_copy(zt_hbm.at[dgi], zd, sem1)
        cp_s.wait()
        cp_d.wait()

        def row(r, rc):
            for k in range(H // 16):
                sv = zs[r, pl.ds(16 * k, 16)]
                dv = zd[r, pl.ds(16 * k, 16)]
                ee = jnp.exp(sv * dv)
                contrib[r, pl.ds(16 * k, 16)] = ee
                contrib[r, pl.ds(H + 16 * k, 16)] = ee * sv
            return rc

        lax.fori_loop(0, CHUNK, row, 0)
        pltpu.sync_copy(contrib, acc_sh.at[didx], add=True)
        return carry

    lax.fori_loop(0, NCH, chunk, 0)
    plsc.subcore_barrier()

    @pl.when(s < TILES_IO)
    def _dump_acc():
        pltpu.sync_copy(acc_sh.at[pl.ds(s * RPT, RPT)],
                        out_hbm.at[pl.ds(c * N + s * RPT, RPT)])


_edge_kernel = pl.kernel(
    _edge_body,
    out_type=jax.ShapeDtypeStruct((2 * N, D), jnp.float32),
    mesh=plsc.VectorSubcoreMesh(core_axis_name="c", subcore_axis_name="s"),
    scratch_types=[
        pltpu.VMEM_SHARED((N, D), jnp.float32),
        pltpu.VMEM((ZR, D), jnp.float32),
        pltpu.VMEM((CHUNK,), jnp.int32),
        pltpu.VMEM((CHUNK,), jnp.int32),
        pltpu.VMEM((CHUNK,), jnp.int32),
        pltpu.VMEM((CHUNK,), jnp.int32),
        pltpu.VMEM((CHUNK, H), jnp.float32),
        pltpu.VMEM((CHUNK, H), jnp.float32),
        pltpu.VMEM((CHUNK, D), jnp.float32),
        pltpu.SemaphoreType.DMA,
        pltpu.SemaphoreType.DMA,
    ],
    compiler_params=pltpu.CompilerParams(use_tc_tiling_on_sc=False),
)


def _post_body(acc_ref, snorm_ref, gamma_ref, beta_ref, out_ref):
    sn = snorm_ref[...]
    for hh in range(2):
        dnm = acc_ref[hh * N:(hh + 1) * N, 0:H]
        num = acc_ref[hh * N:(hh + 1) * N, H:D]
        dnm = jnp.where(dnm == 0.0, 1.0, dnm)
        hagg = num / dnm * sn
        mu = jnp.mean(hagg, axis=0, keepdims=True)
        xc = hagg - mu
        var = jnp.mean(xc * xc, axis=0, keepdims=True)
        g = gamma_ref[0:1, hh * H:(hh + 1) * H]
        b = beta_ref[0:1, hh * H:(hh + 1) * H]
        y = xc * (g * lax.rsqrt(var + EPS)) + b
        out_ref[:, hh * H:(hh + 1) * H] = jnp.where(y > 0, y, jnp.exp(y) - 1.0)


def kernel(h, edge_index, snorm_n, W_fc, gamma, beta):
    src = edge_index[0].astype(jnp.int32)
    dst = edge_index[1].astype(jnp.int32)
    zt = pl.pallas_call(
        _matmul_body,
        out_shape=jax.ShapeDtypeStruct((2 * N, H), jnp.float32),
    )(h, W_fc)
    acc = _edge_kernel(zt, src, dst)
    out = pl.pallas_call(
        _post_body,
        out_shape=jax.ShapeDtypeStruct((N, D), jnp.float32),
    )(acc, snorm_n, gamma.reshape(1, D), beta.reshape(1, D))
    return out


# speedup vs baseline: 5.2978x; 1.4773x over previous
import jax
import jax.numpy as jnp
from jax import lax
from jax.experimental import pallas as pl
from jax.experimental.pallas import tpu as pltpu
from jax.experimental.pallas import tpu_sc as plsc

N = 10000
E = 320000
D = 128
H = 64
EPS = 1e-5

NSUB = 16
EPT = E // NSUB
CHUNK = 80
NCH = EPT // CHUNK
NG = NCH // 2
TILES_IO = 10
RPT = N // TILES_IO
ZR = 8


def _matmul_body(h_ref, w_ref, zlo_ref, zhi_ref):
    z = lax.dot_general(
        h_ref[...], w_ref[...], (((1,), (1,)), ((), ())),
        preferred_element_type=jnp.float32, precision=lax.Precision.HIGHEST)
    zlo_ref[...] = z[:, 0:H]
    zhi_ref[...] = z[:, H:D]


def _edge_body(zlo_hbm, zhi_hbm, src_hbm, dst_hbm, out_hbm,
               acc_sh, zbuf,
               si0, si1, di0, di1, sci0, sci1,
               zs0, zs1, zd0, zd1, ct0, ct1,
               sio0, sio1, sgs0, sgs1, sgd0, sgd1, ssc0, ssc1):
    c = lax.axis_index("c")
    s = lax.axis_index("s")
    ebase = s * EPT

    si = (si0, si1)
    di = (di0, di1)
    sci = (sci0, sci1)
    zs = (zs0, zs1)
    zd = (zd0, zd1)
    ct = (ct0, ct1)
    sio = (sio0, sio1)
    sgs = (sgs0, sgs1)
    sgd = (sgd0, sgd1)
    ssc = (ssc0, ssc1)

    def issue_idx(i, b):
        sl = pl.ds(ebase + i * CHUNK, CHUNK)
        pltpu.async_copy(src_hbm.at[sl], si[b], sio[b])
        pltpu.async_copy(dst_hbm.at[sl], di[b], sio[b])

    def wait_idx(b):
        pltpu.make_async_copy(src_hbm.at[pl.ds(0, CHUNK)], si[b], sio[b]).wait()
        pltpu.make_async_copy(dst_hbm.at[pl.ds(0, CHUNK)], di[b], sio[b]).wait()

    def issue_gather(b):
        @pl.when(c == 0)
        def _lo():
            pltpu.async_copy(zlo_hbm.at[si[b]], zs[b], sgs[b])
            pltpu.async_copy(zlo_hbm.at[di[b]], zd[b], sgd[b])

        @pl.when(c == 1)
        def _hi():
            pltpu.async_copy(zhi_hbm.at[si[b]], zs[b], sgs[b])
            pltpu.async_copy(zhi_hbm.at[di[b]], zd[b], sgd[b])

    def wait_gather(b):
        pltpu.make_async_copy(zlo_hbm.at[si[b]], zs[b], sgs[b]).wait()
        pltpu.make_async_copy(zlo_hbm.at[di[b]], zd[b], sgd[b]).wait()

    def wait_scatter(b):
        pltpu.make_async_copy(ct[b], acc_sh.at[sci[b]], ssc[b]).wait()

    issue_idx(0, 0)
    issue_idx(1, 1)

    @pl.when(s < TILES_IO)
    def _zero_acc():
        def zfill(r, carry):
            for k in range(D // 16):
                zbuf[r, pl.ds(16 * k, 16)] = jnp.zeros((16,), jnp.float32)
            return carry

        lax.fori_loop(0, ZR, zfill, 0)

        def zcopy(j, carry):
            pltpu.sync_copy(zbuf, acc_sh.at[pl.ds(s * RPT + j * ZR, ZR)])
            return carry

        lax.fori_loop(0, RPT // ZR, zcopy, 0)

    wait_idx(0)
    issue_gather(0)
    plsc.subcore_barrier()

    def gloop(g, carry):
        for b in range(2):
            i = g * 2 + b
            bn = 1 - b

            @pl.when(i + 1 < NCH)
            def _advance():
                wait_idx(bn)
                issue_gather(bn)

            wait_gather(b)

            @pl.when(i >= 2)
            def _drain_scatter():
                wait_scatter(b)

            for k in range(CHUNK // 16):
                sl = pl.ds(16 * k, 16)
                sci[b][sl] = di[b][sl]

            @pl.when(i + 2 < NCH)
            def _next_idx():
                issue_idx(i + 2, b)

            def rows(r4, rc):
                for u in range(4):
                    r = r4 * 4 + u
                    for k in range(H // 16):
                        sv = zs[b][r, pl.ds(16 * k, 16)]
                        dv = zd[b][r, pl.ds(16 * k, 16)]
                        ee = jnp.exp(sv * dv)
                        ct[b][r, pl.ds(16 * k, 16)] = ee
                        ct[b][r, pl.ds(H + 16 * k, 16)] = ee * sv
                return rc

            lax.fori_loop(0, CHUNK // 4, rows, 0)

            pltpu.async_copy(ct[b], acc_sh.at[sci[b]], ssc[b], add=True)
        return carry

    lax.fori_loop(0, NG, gloop, 0)
    for b in range(2):
        wait_scatter(b)
    plsc.subcore_barrier()

    @pl.when(s < TILES_IO)
    def _dump_acc():
        pltpu.sync_copy(acc_sh.at[pl.ds(s * RPT, RPT)],
                        out_hbm.at[pl.ds(c * N + s * RPT, RPT)])


_edge_kernel = pl.kernel(
    _edge_body,
    out_type=jax.ShapeDtypeStruct((2 * N, D), jnp.float32),
    mesh=plsc.VectorSubcoreMesh(core_axis_name="c", subcore_axis_name="s"),
    scratch_types=[
        pltpu.VMEM_SHARED((N, D), jnp.float32),
        pltpu.VMEM((ZR, D), jnp.float32),
        pltpu.VMEM((CHUNK,), jnp.int32),
        pltpu.VMEM((CHUNK,), jnp.int32),
        pltpu.VMEM((CHUNK,), jnp.int32),
        pltpu.VMEM((CHUNK,), jnp.int32),
        pltpu.VMEM((CHUNK,), jnp.int32),
        pltpu.VMEM((CHUNK,), jnp.int32),
        pltpu.VMEM((CHUNK, H), jnp.float32),
        pltpu.VMEM((CHUNK, H), jnp.float32),
        pltpu.VMEM((CHUNK, H), jnp.float32),
        pltpu.VMEM((CHUNK, H), jnp.float32),
        pltpu.VMEM((CHUNK, D), jnp.float32),
        pltpu.VMEM((CHUNK, D), jnp.float32),
        pltpu.SemaphoreType.DMA,
        pltpu.SemaphoreType.DMA,
        pltpu.SemaphoreType.DMA,
        pltpu.SemaphoreType.DMA,
        pltpu.SemaphoreType.DMA,
        pltpu.SemaphoreType.DMA,
        pltpu.SemaphoreType.DMA,
        pltpu.SemaphoreType.DMA,
    ],
    compiler_params=pltpu.CompilerParams(use_tc_tiling_on_sc=False),
)


def _post_body(acc_ref, snorm_ref, gamma_ref, beta_ref, out_ref):
    sn = snorm_ref[...]
    for hh in range(2):
        dnm = acc_ref[hh * N:(hh + 1) * N, 0:H]
        num = acc_ref[hh * N:(hh + 1) * N, H:D]
        dnm = jnp.where(dnm == 0.0, 1.0, dnm)
        hagg = num / dnm * sn
        mu = jnp.mean(hagg, axis=0, keepdims=True)
        xc = hagg - mu
        var = jnp.mean(xc * xc, axis=0, keepdims=True)
        g = gamma_ref[0:1, hh * H:(hh + 1) * H]
        b = beta_ref[0:1, hh * H:(hh + 1) * H]
        y = xc * (g * lax.rsqrt(var + EPS)) + b
        out_ref[:, hh * H:(hh + 1) * H] = jnp.where(y > 0, y, jnp.exp(y) - 1.0)


def kernel(h, edge_index, snorm_n, W_fc, gamma, beta):
    src = edge_index[0].astype(jnp.int32)
    dst = edge_index[1].astype(jnp.int32)
    z_lo, z_hi = pl.pallas_call(
        _matmul_body,
        out_shape=[jax.ShapeDtypeStruct((N, H), jnp.float32),
                   jax.ShapeDtypeStruct((N, H), jnp.float32)],
    )(h, W_fc)
    acc = _edge_kernel(z_lo, z_hi, src, dst)
    out = pl.pallas_call(
        _post_body,
        out_shape=jax.ShapeDtypeStruct((N, D), jnp.float32),
    )(acc, snorm_n, gamma.reshape(1, D), beta.reshape(1, D))
    return out


# speedup vs baseline: 18.6555x; 3.5214x over previous
import jax
import jax.numpy as jnp
from jax import lax
from jax.experimental import pallas as pl
from jax.experimental.pallas import tpu as pltpu
from jax.experimental.pallas import tpu_sc as plsc

N = 10000
E = 320000
D = 128
H = 64
EPS = 1e-5

NSUB = 16
EPT = E // NSUB
CHUNK = 80
NCH = EPT // CHUNK
NG = NCH // 2
TILES_IO = 10
RPT = N // TILES_IO
ZR = 8


def _matmul_body(h_ref, w_ref, zlo_ref, zhi_ref):
    z = lax.dot_general(
        h_ref[...], w_ref[...], (((1,), (1,)), ((), ())),
        preferred_element_type=jnp.float32, precision=lax.Precision.HIGHEST)
    zlo_ref[...] = z[:, 0:H]
    zhi_ref[...] = z[:, H:D]


def _edge_body(zlo_hbm, zhi_hbm, src_hbm, dst_hbm, out_hbm,
               acc_sh, zbuf,
               si0, si1, di0, di1, sci0, sci1,
               zs0, zs1, zd0, zd1, ct0, ct1,
               sio0, sio1, sgs0, sgs1, sgd0, sgd1, ssc0, ssc1):
    c = lax.axis_index("c")
    s = lax.axis_index("s")
    ebase = s * EPT

    si = (si0, si1)
    di = (di0, di1)
    sci = (sci0, sci1)
    zs = (zs0, zs1)
    zd = (zd0, zd1)
    ct = (ct0, ct1)
    sio = (sio0, sio1)
    sgs = (sgs0, sgs1)
    sgd = (sgd0, sgd1)
    ssc = (ssc0, ssc1)

    def issue_idx(i, b):
        sl = pl.ds(ebase + i * CHUNK, CHUNK)
        pltpu.async_copy(src_hbm.at[sl], si[b], sio[b])
        pltpu.async_copy(dst_hbm.at[sl], di[b], sio[b])

    def wait_idx(b):
        pltpu.make_async_copy(src_hbm.at[pl.ds(0, CHUNK)], si[b], sio[b]).wait()
        pltpu.make_async_copy(dst_hbm.at[pl.ds(0, CHUNK)], di[b], sio[b]).wait()

    def issue_gather(b):
        @pl.when(c == 0)
        def _lo():
            pltpu.async_copy(zlo_hbm.at[si[b]], zs[b], sgs[b])
            pltpu.async_copy(zlo_hbm.at[di[b]], zd[b], sgd[b])

        @pl.when(c == 1)
        def _hi():
            pltpu.async_copy(zhi_hbm.at[si[b]], zs[b], sgs[b])
            pltpu.async_copy(zhi_hbm.at[di[b]], zd[b], sgd[b])

    def wait_gather(b):
        pltpu.make_async_copy(zlo_hbm.at[si[b]], zs[b], sgs[b]).wait()
        pltpu.make_async_copy(zlo_hbm.at[di[b]], zd[b], sgd[b]).wait()

    def wait_scatter(b):
        pltpu.make_async_copy(ct[b], acc_sh.at[sci[b]], ssc[b]).wait()

    issue_idx(0, 0)
    issue_idx(1, 1)

    @pl.when(s < TILES_IO)
    def _zero_acc():
        def zfill(r, carry):
            for k in range(D // 16):
                zbuf[r, pl.ds(16 * k, 16)] = jnp.zeros((16,), jnp.float32)
            return carry

        lax.fori_loop(0, ZR, zfill, 0)

        def zcopy(j, carry):
            pltpu.sync_copy(zbuf, acc_sh.at[pl.ds(s * RPT + j * ZR, ZR)])
            return carry

        lax.fori_loop(0, RPT // ZR, zcopy, 0)

    wait_idx(0)
    issue_gather(0)
    plsc.subcore_barrier()

    def gloop(g, carry):
        for b in range(2):
            i = g * 2 + b
            bn = 1 - b

            @pl.when(i + 1 < NCH)
            def _advance():
                wait_idx(bn)
                issue_gather(bn)

            wait_gather(b)

            @pl.when(i >= 2)
            def _drain_scatter():
                wait_scatter(b)

            for k in range(CHUNK // 16):
                sl = pl.ds(16 * k, 16)
                sci[b][sl] = di[b][sl]

            @pl.when(i + 2 < NCH)
            def _next_idx():
                issue_idx(i + 2, b)

            @plsc.parallel_loop(0, CHUNK, step=1, unroll=4)
            def _rows(r):
                for k in range(H // 16):
                    sv = zs[b][r, pl.ds(16 * k, 16)]
                    dv = zd[b][r, pl.ds(16 * k, 16)]
                    ee = jnp.exp(sv * dv)
                    ct[b][r, pl.ds(16 * k, 16)] = ee
                    ct[b][r, pl.ds(H + 16 * k, 16)] = ee * sv

            pltpu.async_copy(ct[b], acc_sh.at[sci[b]], ssc[b], add=True)
        return carry

    lax.fori_loop(0, NG, gloop, 0)
    for b in range(2):
        wait_scatter(b)
    plsc.subcore_barrier()

    @pl.when(s < TILES_IO)
    def _dump_acc():
        pltpu.sync_copy(acc_sh.at[pl.ds(s * RPT, RPT)],
                        out_hbm.at[pl.ds(c * N + s * RPT, RPT)])


_edge_kernel = pl.kernel(
    _edge_body,
    out_type=jax.ShapeDtypeStruct((2 * N, D), jnp.float32),
    mesh=plsc.VectorSubcoreMesh(core_axis_name="c", subcore_axis_name="s"),
    scratch_types=[
        pltpu.VMEM_SHARED((N, D), jnp.float32),
        pltpu.VMEM((ZR, D), jnp.float32),
        pltpu.VMEM((CHUNK,), jnp.int32),
        pltpu.VMEM((CHUNK,), jnp.int32),
        pltpu.VMEM((CHUNK,), jnp.int32),
        pltpu.VMEM((CHUNK,), jnp.int32),
        pltpu.VMEM((CHUNK,), jnp.int32),
        pltpu.VMEM((CHUNK,), jnp.int32),
        pltpu.VMEM((CHUNK, H), jnp.float32),
        pltpu.VMEM((CHUNK, H), jnp.float32),
        pltpu.VMEM((CHUNK, H), jnp.float32),
        pltpu.VMEM((CHUNK, H), jnp.float32),
        pltpu.VMEM((CHUNK, D), jnp.float32),
        pltpu.VMEM((CHUNK, D), jnp.float32),
        pltpu.SemaphoreType.DMA,
        pltpu.SemaphoreType.DMA,
        pltpu.SemaphoreType.DMA,
        pltpu.SemaphoreType.DMA,
        pltpu.SemaphoreType.DMA,
        pltpu.SemaphoreType.DMA,
        pltpu.SemaphoreType.DMA,
        pltpu.SemaphoreType.DMA,
    ],
    compiler_params=pltpu.CompilerParams(use_tc_tiling_on_sc=False),
)


def _post_body(acc_ref, snorm_ref, gamma_ref, beta_ref, out_ref):
    sn = snorm_ref[...]
    for hh in range(2):
        dnm = acc_ref[hh * N:(hh + 1) * N, 0:H]
        num = acc_ref[hh * N:(hh + 1) * N, H:D]
        dnm = jnp.where(dnm == 0.0, 1.0, dnm)
        hagg = num / dnm * sn
        mu = jnp.mean(hagg, axis=0, keepdims=True)
        xc = hagg - mu
        var = jnp.mean(xc * xc, axis=0, keepdims=True)
        g = gamma_ref[0:1, hh * H:(hh + 1) * H]
        b = beta_ref[0:1, hh * H:(hh + 1) * H]
        y = xc * (g * lax.rsqrt(var + EPS)) + b
        out_ref[:, hh * H:(hh + 1) * H] = jnp.where(y > 0, y, jnp.exp(y) - 1.0)


def kernel(h, edge_index, snorm_n, W_fc, gamma, beta):
    src = edge_index[0].astype(jnp.int32)
    dst = edge_index[1].astype(jnp.int32)
    z_lo, z_hi = pl.pallas_call(
        _matmul_body,
        out_shape=[jax.ShapeDtypeStruct((N, H), jnp.float32),
                   jax.ShapeDtypeStruct((N, H), jnp.float32)],
    )(h, W_fc)
    acc = _edge_kernel(z_lo, z_hi, src, dst)
    out = pl.pallas_call(
        _post_body,
        out_shape=jax.ShapeDtypeStruct((N, D), jnp.float32),
    )(acc, snorm_n, gamma.reshape(1, D), beta.reshape(1, D))
    return out
